# baseline (speedup 1.0000x reference)
import functools

import jax
import jax.numpy as jnp
from jax import lax
from jax.experimental import pallas as pl
from jax.experimental.pallas import tpu as pltpu
from jax.experimental.pallas import tpu_sc as plsc

_N, _E, _D, _C = 10000, 320000, 128, 7
_NC, _NS = 2, 16
_NW = _NC * _NS
_RANGE = 313
_NPAD = _NW * _RANGE
_W = 4000
_G = 128
_DUMP = _W + _G


def _sc_body(x_hbm, src_hbm, dst_hbm, out_hbm,
             src_w, dst_w, src_sel, dst_sel, rows, acc, sem):
    c = lax.axis_index("c")
    s = lax.axis_index("s")
    wid = s * _NC + c
    lo = wid * _RANGE

    neg16 = jnp.full((16,), -jnp.inf, dtype=jnp.float32)

    def init_body(i, carry):
        acc[pl.ds(i * 16, 16)] = neg16
        return carry

    lax.fori_loop(0, ((_RANGE + 1) * _D) // 16, init_body, 0)

    def win_body(w, carry):
        base = w * _W
        pltpu.sync_copy(src_hbm.at[pl.ds(base, _W)], src_w)
        pltpu.sync_copy(dst_hbm.at[pl.ds(base, _W)], dst_w)

        def scan_body(i, cnt):
            d16 = dst_w[pl.ds(i * 16, 16)]
            s16 = src_w[pl.ds(i * 16, 16)]
            m = (d16 >= lo) & (d16 < lo + _RANGE)
            pos = plsc.cumsum(m.astype(jnp.int32))
            idx = jnp.where(m, cnt + pos - 1, _DUMP)
            plsc.store_scatter(dst_sel, [idx], d16 - lo)
            plsc.store_scatter(src_sel, [idx], s16)
            return cnt + pos[15]

        cnt = lax.fori_loop(0, _W // 16, scan_body, 0)

        zero16 = jnp.zeros((16,), dtype=jnp.int32)
        trash16 = jnp.full((16,), _RANGE, dtype=jnp.int32)
        for t in range(_G // 16):
            src_sel[pl.ds(cnt + t * 16, 16)] = zero16
            dst_sel[pl.ds(cnt + t * 16, 16)] = trash16

        nch = (cnt + _G - 1) // _G

        def chunk_body(ci, carry2):
            cbase = ci * _G
            pltpu.async_copy(
                x_hbm.at[src_sel.at[pl.ds(cbase, _G)]], rows, sem).wait()

            def grp_body(g, carry3):
                d16v = dst_sel[pl.ds(cbase + g * 16, 16)]
                for jj in range(16):
                    abase = d16v[jj] * _D
                    r = g * 16 + jj
                    for f in range(_D // 16):
                        sl = pl.ds(abase + f * 16, 16)
                        acc[sl] = jnp.maximum(acc[sl], rows[r, f * 16:(f + 1) * 16])
                return carry3

            lax.fori_loop(0, _G // 16, grp_body, 0)
            return carry2

        lax.fori_loop(0, nch, chunk_body, 0)
        return carry

    lax.fori_loop(0, _E // _W, win_body, 0)

    pltpu.sync_copy(acc.at[pl.ds(0, _RANGE * _D)],
                    out_hbm.at[pl.ds(lo * _D, _RANGE * _D)])


def _segment_max_sc(x, src, dst):
    mesh = plsc.VectorSubcoreMesh(core_axis_name="c", subcore_axis_name="s")
    f = pl.kernel(
        _sc_body,
        out_type=jax.ShapeDtypeStruct((_NPAD * _D,), jnp.float32),
        mesh=mesh,
        compiler_params=pltpu.CompilerParams(needs_layout_passes=False),
        scratch_types=[
            pltpu.VMEM((_W,), jnp.int32),
            pltpu.VMEM((_W,), jnp.int32),
            pltpu.VMEM((_W + _G + 16,), jnp.int32),
            pltpu.VMEM((_W + _G + 16,), jnp.int32),
            pltpu.VMEM((_G, _D), jnp.float32),
            pltpu.VMEM(((_RANGE + 1) * _D,), jnp.float32),
            pltpu.SemaphoreType.DMA,
        ],
    )
    return f(x, src, dst)


def _tc_body(x_ref, agg_ref, wl_ref, b_ref, wr_ref, o_ref):
    a = agg_ref[...]
    a = jnp.where(a == -jnp.inf, 0.0, a)
    z = (jnp.dot(a, wl_ref[...], preferred_element_type=jnp.float32)
         + jnp.dot(x_ref[...], wr_ref[...], preferred_element_type=jnp.float32)
         + b_ref[...])
    col = lax.broadcasted_iota(jnp.int32, z.shape, 1)
    z = jnp.where(col < _C, z, -jnp.inf)
    m = jnp.max(z, axis=1, keepdims=True)
    zs = z - m
    lse = jnp.log(jnp.sum(jnp.exp(zs), axis=1, keepdims=True))
    o_ref[...] = zs - lse


_BN = 400


def _dense_tc(x, agg, wl, b, wr):
    return pl.pallas_call(
        _tc_body,
        grid=(_N // _BN,),
        in_specs=[
            pl.BlockSpec((_BN, _D), lambda i: (i, 0)),
            pl.BlockSpec((_BN, _D), lambda i: (i, 0)),
            pl.BlockSpec((_D, 128), lambda i: (0, 0)),
            pl.BlockSpec((1, 128), lambda i: (0, 0)),
            pl.BlockSpec((_D, 128), lambda i: (0, 0)),
        ],
        out_specs=pl.BlockSpec((_BN, 128), lambda i: (i, 0)),
        out_shape=jax.ShapeDtypeStruct((_N, 128), jnp.float32),
    )(x, agg, wl, b, wr)


def kernel(x, edge_index, W_l, b_l, W_r):
    src = edge_index[0]
    dst = edge_index[1]
    aggf = _segment_max_sc(x, src, dst)
    agg = aggf.reshape(_NPAD, _D)
    wl = jnp.zeros((_D, 128), jnp.float32).at[:, :_C].set(W_l.T)
    wr = jnp.zeros((_D, 128), jnp.float32).at[:, :_C].set(W_r.T)
    b = jnp.zeros((1, 128), jnp.float32).at[0, :_C].set(b_l)
    out = _dense_tc(x, agg, wl, b, wr)
    return out[:, :_C]


# speedup vs baseline: 1.9507x; 1.9507x over previous
import functools

import jax
import jax.numpy as jnp
from jax import lax
from jax.experimental import pallas as pl
from jax.experimental.pallas import tpu as pltpu
from jax.experimental.pallas import tpu_sc as plsc

_N, _E, _D, _C = 10000, 320000, 128, 7
_NC, _NS = 2, 16
_NW = _NC * _NS
_RANGE = 313
_NPAD = _NW * _RANGE
_W = 8000
_G = 128
_UNROLL = 4
_DUMP = _W + _G


def _sc_body(x_hbm, src_hbm, dst_hbm, out_hbm,
             src_w, dst_w, src_sel, dst_sel, rows0, rows1, acc, sem0, sem1):
    c = lax.axis_index("c")
    s = lax.axis_index("s")
    wid = s * _NC + c
    lo = wid * _RANGE

    neg16 = jnp.full((16,), -jnp.inf, dtype=jnp.float32)
    rng16 = jnp.full((16,), _RANGE, dtype=jnp.uint32)
    one16 = jnp.ones((16,), dtype=jnp.int32)
    zero16 = jnp.zeros((16,), dtype=jnp.int32)
    trash16 = jnp.full((16,), _RANGE, dtype=jnp.int32)

    def init_body(i, carry):
        acc[pl.ds(i * 16, 16)] = neg16
        return carry

    lax.fori_loop(0, ((_RANGE + 1) * _D) // 16, init_body, 0)

    def win_body(w, carry):
        base = w * _W
        pltpu.sync_copy(src_hbm.at[pl.ds(base, _W)], src_w)
        pltpu.sync_copy(dst_hbm.at[pl.ds(base, _W)], dst_w)

        def scan_body(i, cnt):
            for u in range(_UNROLL):
                off = (i * _UNROLL + u) * 16
                d16 = dst_w[pl.ds(off, 16)]
                s16 = src_w[pl.ds(off, 16)]
                du = d16 - lo
                m = plsc.bitcast(du, jnp.uint32) < rng16
                pos = plsc.cumsum(jnp.where(m, one16, zero16))
                idx = jnp.where(m, cnt + pos - 1, _DUMP)
                plsc.store_scatter(dst_sel, [idx], du)
                plsc.store_scatter(src_sel, [idx], s16)
                cnt = cnt + pos[15]
            return cnt

        cnt = lax.fori_loop(0, _W // (16 * _UNROLL), scan_body, 0)

        for t in range(_G // 16):
            src_sel[pl.ds(cnt + t * 16, 16)] = zero16
            dst_sel[pl.ds(cnt + t * 16, 16)] = trash16

        nch = (cnt + _G - 1) // _G

        @pl.when(nch > 0)
        def _():
            pltpu.async_copy(x_hbm.at[src_sel.at[pl.ds(0, _G)]], rows0, sem0)

        def rmw(rows, cbase):
            def grp_body(g, carry3):
                d16v = dst_sel[pl.ds(cbase + g * 16, 16)]
                for jj in range(16):
                    abase = d16v[jj] * _D
                    r = g * 16 + jj
                    for f in range(_D // 16):
                        sl = pl.ds(abase + f * 16, 16)
                        acc[sl] = jnp.maximum(acc[sl], rows[r, f * 16:(f + 1) * 16])
                return carry3

            lax.fori_loop(0, _G // 16, grp_body, 0)

        def step(cur, csem, nxt, nsem, ci):
            @pl.when(ci + 1 < nch)
            def _():
                pltpu.async_copy(
                    x_hbm.at[src_sel.at[pl.ds((ci + 1) * _G, _G)]], nxt, nsem)
            pltpu.make_async_copy(
                x_hbm.at[src_sel.at[pl.ds(ci * _G, _G)]], cur, csem).wait()
            rmw(cur, ci * _G)

        def chunk_body(ci, carry2):
            @pl.when(ci % 2 == 0)
            def _():
                step(rows0, sem0, rows1, sem1, ci)

            @pl.when(ci % 2 == 1)
            def _():
                step(rows1, sem1, rows0, sem0, ci)
            return carry2

        lax.fori_loop(0, nch, chunk_body, 0)
        return carry

    lax.fori_loop(0, _E // _W, win_body, 0)

    pltpu.sync_copy(acc.at[pl.ds(0, _RANGE * _D)],
                    out_hbm.at[pl.ds(lo * _D, _RANGE * _D)])


def _segment_max_sc(x, src, dst):
    mesh = plsc.VectorSubcoreMesh(core_axis_name="c", subcore_axis_name="s")
    f = pl.kernel(
        _sc_body,
        out_type=jax.ShapeDtypeStruct((_NPAD * _D,), jnp.float32),
        mesh=mesh,
        compiler_params=pltpu.CompilerParams(needs_layout_passes=False),
        scratch_types=[
            pltpu.VMEM((_W,), jnp.int32),
            pltpu.VMEM((_W,), jnp.int32),
            pltpu.VMEM((_W + _G + 16,), jnp.int32),
            pltpu.VMEM((_W + _G + 16,), jnp.int32),
            pltpu.VMEM((_G, _D), jnp.float32),
            pltpu.VMEM((_G, _D), jnp.float32),
            pltpu.VMEM(((_RANGE + 1) * _D,), jnp.float32),
            pltpu.SemaphoreType.DMA,
            pltpu.SemaphoreType.DMA,
        ],
    )
    return f(x, src, dst)


def _tc_body(x_ref, agg_ref, wl_ref, b_ref, wr_ref, o_ref):
    a = agg_ref[...]
    a = jnp.where(a == -jnp.inf, 0.0, a)
    z = (jnp.dot(a, wl_ref[...], preferred_element_type=jnp.float32)
         + jnp.dot(x_ref[...], wr_ref[...], preferred_element_type=jnp.float32)
         + b_ref[...])
    col = lax.broadcasted_iota(jnp.int32, z.shape, 1)
    z = jnp.where(col < _C, z, -jnp.inf)
    m = jnp.max(z, axis=1, keepdims=True)
    zs = z - m
    lse = jnp.log(jnp.sum(jnp.exp(zs), axis=1, keepdims=True))
    o_ref[...] = zs - lse


_BN = 400


def _dense_tc(x, agg, wl, b, wr):
    return pl.pallas_call(
        _tc_body,
        grid=(_N // _BN,),
        in_specs=[
            pl.BlockSpec((_BN, _D), lambda i: (i, 0)),
            pl.BlockSpec((_BN, _D), lambda i: (i, 0)),
            pl.BlockSpec((_D, 128), lambda i: (0, 0)),
            pl.BlockSpec((1, 128), lambda i: (0, 0)),
            pl.BlockSpec((_D, 128), lambda i: (0, 0)),
        ],
        out_specs=pl.BlockSpec((_BN, 128), lambda i: (i, 0)),
        out_shape=jax.ShapeDtypeStruct((_N, 128), jnp.float32),
    )(x, agg, wl, b, wr)


def kernel(x, edge_index, W_l, b_l, W_r):
    src = edge_index[0]
    dst = edge_index[1]
    aggf = _segment_max_sc(x, src, dst)
    agg = aggf.reshape(_NPAD, _D)
    wl = jnp.zeros((_D, 128), jnp.float32).at[:, :_C].set(W_l.T)
    wr = jnp.zeros((_D, 128), jnp.float32).at[:, :_C].set(W_r.T)
    b = jnp.zeros((1, 128), jnp.float32).at[0, :_C].set(b_l)
    out = _dense_tc(x, agg, wl, b, wr)
    return out[:, :_C]


# speedup vs baseline: 1.9634x; 1.0065x over previous
import functools

import jax
import jax.numpy as jnp
from jax import lax
from jax.experimental import pallas as pl
from jax.experimental.pallas import tpu as pltpu
from jax.experimental.pallas import tpu_sc as plsc

_N, _E, _D, _C = 10000, 320000, 128, 7
_NC, _NS = 2, 16
_NW = _NC * _NS
_RANGE = 313
_NPAD = _NW * _RANGE
_W = 8000
_G = 128
_UNROLL = 4
_DUMP = _W + _G


def _sc_body(x_hbm, src_hbm, dst_hbm, out_hbm,
             src_w, dst_w, src_sel, dst_sel, rows0, rows1, acc,
             sem0, sem1):
    c = lax.axis_index("c")
    s = lax.axis_index("s")
    wid = s * _NC + c
    lo = wid * _RANGE

    neg16 = jnp.full((16,), -jnp.inf, dtype=jnp.float32)
    rng16 = jnp.full((16,), _RANGE, dtype=jnp.uint32)
    one16 = jnp.ones((16,), dtype=jnp.int32)
    zero16 = jnp.zeros((16,), dtype=jnp.int32)
    trash16 = jnp.full((16,), _RANGE, dtype=jnp.int32)
    lane15 = jnp.full((16,), 15, dtype=jnp.int32)

    def init_body(i, carry):
        acc[pl.ds(i * 16, 16)] = neg16
        return carry

    lax.fori_loop(0, ((_RANGE + 1) * _D) // 16, init_body, 0)

    def win_body(w, carry):
        base = w * _W
        pltpu.sync_copy(src_hbm.at[pl.ds(base, _W)], src_w)
        pltpu.sync_copy(dst_hbm.at[pl.ds(base, _W)], dst_w)

        def scan_body(i, cntv):
            for u in range(_UNROLL):
                off = (i * _UNROLL + u) * 16
                d16 = dst_w[pl.ds(off, 16)]
                s16 = src_w[pl.ds(off, 16)]
                du = d16 - lo
                m = plsc.bitcast(du, jnp.uint32) < rng16
                pos = plsc.cumsum(jnp.where(m, one16, zero16))
                idx = jnp.where(m, cntv + pos - 1, _DUMP)
                plsc.store_scatter(dst_sel, [idx], du)
                plsc.store_scatter(src_sel, [idx], s16)
                cntv = cntv + pos.at[lane15].get(mode="promise_in_bounds")
            return cntv

        cntv = lax.fori_loop(0, _W // (16 * _UNROLL), scan_body, zero16)
        cnt = cntv[0]

        for t in range(_G // 16):
            src_sel[pl.ds(cnt + t * 16, 16)] = zero16
            dst_sel[pl.ds(cnt + t * 16, 16)] = trash16

        nch = (cnt + _G - 1) // _G

        @pl.when(nch > 0)
        def _():
            pltpu.async_copy(x_hbm.at[src_sel.at[pl.ds(0, _G)]], rows0, sem0)

        def rmw(rows, cbase):
            def grp_body(g, carry3):
                d16v = dst_sel[pl.ds(cbase + g * 16, 16)]
                offv = d16v * _D
                dls = [offv[jj] for jj in range(16)]
                for jj in range(16):
                    abase = dls[jj]
                    r = g * 16 + jj
                    rv = [rows[r, f * 16:(f + 1) * 16] for f in range(8)]
                    av = [acc[pl.ds(abase + f * 16, 16)] for f in range(8)]
                    mx = [jnp.maximum(av[f], rv[f]) for f in range(8)]
                    for f in range(8):
                        acc[pl.ds(abase + f * 16, 16)] = mx[f]
                return carry3

            lax.fori_loop(0, _G // 16, grp_body, 0)

        def step(cur, csem, nxt, nsem, ci):
            @pl.when(ci + 1 < nch)
            def _():
                pltpu.async_copy(
                    x_hbm.at[src_sel.at[pl.ds((ci + 1) * _G, _G)]], nxt, nsem)
            pltpu.make_async_copy(
                x_hbm.at[src_sel.at[pl.ds(ci * _G, _G)]], cur, csem).wait()
            rmw(cur, ci * _G)

        def chunk_body(ci, carry2):
            @pl.when(ci % 2 == 0)
            def _():
                step(rows0, sem0, rows1, sem1, ci)

            @pl.when(ci % 2 == 1)
            def _():
                step(rows1, sem1, rows0, sem0, ci)
            return carry2

        lax.fori_loop(0, nch, chunk_body, 0)
        return carry

    lax.fori_loop(0, _E // _W, win_body, 0)

    pltpu.sync_copy(acc.at[pl.ds(0, _RANGE * _D)],
                    out_hbm.at[pl.ds(lo * _D, _RANGE * _D)])


def _segment_max_sc(x, src, dst):
    mesh = plsc.VectorSubcoreMesh(core_axis_name="c", subcore_axis_name="s")
    f = pl.kernel(
        _sc_body,
        out_type=jax.ShapeDtypeStruct((_NPAD * _D,), jnp.float32),
        mesh=mesh,
        compiler_params=pltpu.CompilerParams(needs_layout_passes=False),
        scratch_types=[
            pltpu.VMEM((_W,), jnp.int32),
            pltpu.VMEM((_W,), jnp.int32),
            pltpu.VMEM((_W + _G + 16,), jnp.int32),
            pltpu.VMEM((_W + _G + 16,), jnp.int32),
            pltpu.VMEM((_G, _D), jnp.float32),
            pltpu.VMEM((_G, _D), jnp.float32),
            pltpu.VMEM(((_RANGE + 1) * _D,), jnp.float32),
            pltpu.SemaphoreType.DMA,
            pltpu.SemaphoreType.DMA,
        ],
    )
    return f(x, src, dst)


def _tc_body(x_ref, agg_ref, wl_ref, b_ref, wr_ref, o_ref):
    a = agg_ref[...]
    a = jnp.where(a == -jnp.inf, 0.0, a)
    z = (jnp.dot(a, wl_ref[...], preferred_element_type=jnp.float32)
         + jnp.dot(x_ref[...], wr_ref[...], preferred_element_type=jnp.float32)
         + b_ref[...])
    col = lax.broadcasted_iota(jnp.int32, z.shape, 1)
    z = jnp.where(col < _C, z, -jnp.inf)
    m = jnp.max(z, axis=1, keepdims=True)
    zs = z - m
    lse = jnp.log(jnp.sum(jnp.exp(zs), axis=1, keepdims=True))
    o_ref[...] = zs - lse


_BN = 400


def _dense_tc(x, agg, wl, b, wr):
    return pl.pallas_call(
        _tc_body,
        grid=(_N // _BN,),
        in_specs=[
            pl.BlockSpec((_BN, _D), lambda i: (i, 0)),
            pl.BlockSpec((_BN, _D), lambda i: (i, 0)),
            pl.BlockSpec((_D, 128), lambda i: (0, 0)),
            pl.BlockSpec((1, 128), lambda i: (0, 0)),
            pl.BlockSpec((_D, 128), lambda i: (0, 0)),
        ],
        out_specs=pl.BlockSpec((_BN, 128), lambda i: (i, 0)),
        out_shape=jax.ShapeDtypeStruct((_N, 128), jnp.float32),
    )(x, agg, wl, b, wr)


def kernel(x, edge_index, W_l, b_l, W_r):
    src = edge_index[0]
    dst = edge_index[1]
    aggf = _segment_max_sc(x, src, dst)
    agg = aggf.reshape(_NPAD, _D)
    wl = jnp.zeros((_D, 128), jnp.float32).at[:, :_C].set(W_l.T)
    wr = jnp.zeros((_D, 128), jnp.float32).at[:, :_C].set(W_r.T)
    b = jnp.zeros((1, 128), jnp.float32).at[0, :_C].set(b_l)
    out = _dense_tc(x, agg, wl, b, wr)
    return out[:, :_C]
